# Initial kernel scaffold; baseline (speedup 1.0000x reference)
import jax
import jax.numpy as jnp
from jax.experimental import pallas as pl


def kernel(fMRI, W1, b1, W2, b2):
    raise NotImplementedError("write your pallas kernel here")


# speedup vs baseline: 7.4921x; 7.4921x over previous
import functools

import jax
import jax.numpy as jnp
from jax.experimental import pallas as pl

B, T, N, H = 8, 512, 400, 128
EPS = 1e-8


def _gcn_block_kernel(fmri_ref, w1_ref, b1_ref, w2_ref, b2_ref, out_ref):
    x = fmri_ref[0]
    mean = jnp.mean(x, axis=0, keepdims=True)
    xc = x - mean
    cov = jax.lax.dot_general(
        xc, xc, (((0,), (0,)), ((), ())),
        preferred_element_type=jnp.float32)
    var = jnp.sum(xc * xc, axis=0)
    s = jnp.sqrt(var)
    den = s[:, None] * s[None, :]
    corr = jnp.where(den > 0.0, cov / den, 0.0)
    adj = jnp.abs(jnp.clip(corr, -1.0, 1.0))

    rown = jnp.sqrt(jnp.sum(adj * adj, axis=1, keepdims=True))
    a = adj / (rown + EPS)
    deg = jnp.sum(a, axis=1, keepdims=True) + 1.0
    dinv = jax.lax.rsqrt(deg)

    def layer(xin, w, bias):
        sup = jnp.dot(xin, w, preferred_element_type=jnp.float32) + bias
        sup = sup * dinv
        agg = jnp.dot(a, sup, preferred_element_type=jnp.float32) + sup
        return jnp.maximum(agg * dinv, 0.0)

    x1 = layer(a, w1_ref[...], b1_ref[...])
    out_ref[0] = layer(x1, w2_ref[...], b2_ref[...])


@jax.jit
def kernel(fMRI, W1, b1, W2, b2):
    b1r = b1.reshape(1, H)
    b2r = b2.reshape(1, H)
    grid = (B,)
    out = pl.pallas_call(
        _gcn_block_kernel,
        grid=grid,
        in_specs=[
            pl.BlockSpec((1, T, N), lambda b: (b, 0, 0)),
            pl.BlockSpec((N, H), lambda b: (0, 0)),
            pl.BlockSpec((1, H), lambda b: (0, 0)),
            pl.BlockSpec((H, H), lambda b: (0, 0)),
            pl.BlockSpec((1, H), lambda b: (0, 0)),
        ],
        out_specs=pl.BlockSpec((1, N, H), lambda b: (b, 0, 0)),
        out_shape=jax.ShapeDtypeStruct((B, N, H), jnp.float32),
    )(fMRI, W1, b1r, W2, b2r)
    return out


# speedup vs baseline: 7.8884x; 1.0529x over previous
import jax
import jax.numpy as jnp
from jax.experimental import pallas as pl

B, T, N, H = 8, 512, 400, 128
EPS = 1e-8


def _gcn_block_kernel(fmri_ref, w1_ref, b1_ref, w2_ref, b2_ref, out_ref):
    x = fmri_ref[0]
    mean = jnp.mean(x, axis=0, keepdims=True)
    xc = x - mean
    cov = jax.lax.dot_general(
        xc, xc, (((0,), (0,)), ((), ())),
        preferred_element_type=jnp.float32)
    var = jnp.sum(xc * xc, axis=0)
    s = jnp.sqrt(var)
    inv_s = jnp.where(s > 0.0, 1.0 / s, 0.0)
    corr = cov * inv_s[:, None] * inv_s[None, :]
    adj = jnp.abs(jnp.clip(corr, -1.0, 1.0))

    rs1 = jnp.sum(adj, axis=1, keepdims=True)
    rs2 = jnp.sum(adj * adj, axis=1, keepdims=True)
    inv_rn = 1.0 / (jnp.sqrt(rs2) + EPS)
    deg = rs1 * inv_rn + 1.0
    dinv = jax.lax.rsqrt(deg)

    def layer(sup):
        supd = sup * dinv
        agg = inv_rn * jnp.dot(adj, supd, preferred_element_type=jnp.float32)
        return jnp.maximum((agg + supd) * dinv, 0.0)

    w1 = w1_ref[...]
    s1 = inv_rn * jnp.dot(adj, w1, preferred_element_type=jnp.float32)
    x1 = layer(s1 + b1_ref[...])
    s2 = jnp.dot(x1, w2_ref[...], preferred_element_type=jnp.float32)
    out_ref[0] = layer(s2 + b2_ref[...])


@jax.jit
def kernel(fMRI, W1, b1, W2, b2):
    b1r = b1.reshape(1, H)
    b2r = b2.reshape(1, H)
    out = pl.pallas_call(
        _gcn_block_kernel,
        grid=(B,),
        in_specs=[
            pl.BlockSpec((1, T, N), lambda b: (b, 0, 0)),
            pl.BlockSpec((N, H), lambda b: (0, 0)),
            pl.BlockSpec((1, H), lambda b: (0, 0)),
            pl.BlockSpec((H, H), lambda b: (0, 0)),
            pl.BlockSpec((1, H), lambda b: (0, 0)),
        ],
        out_specs=pl.BlockSpec((1, N, H), lambda b: (b, 0, 0)),
        out_shape=jax.ShapeDtypeStruct((B, N, H), jnp.float32),
    )(fMRI, W1, b1r, W2, b2r)
    return out


# speedup vs baseline: 7.9064x; 1.0023x over previous
import jax
import jax.numpy as jnp
from jax.experimental import pallas as pl

B, T, N, H = 8, 512, 400, 128
EPS = 1e-8


def _gcn_block_kernel(fmri_ref, w1_ref, b1_ref, w2_ref, b2_ref, out_ref):
    x = fmri_ref[0]
    xb = x.astype(jnp.bfloat16)
    gram = jax.lax.dot_general(
        xb, xb, (((0,), (0,)), ((), ())),
        preferred_element_type=jnp.float32)
    colsum = jnp.sum(x, axis=0)
    sumsq = jnp.sum(x * x, axis=0)
    m = colsum * (1.0 / T)
    var = sumsq - T * m * m
    s = jnp.sqrt(var)
    inv_s = jnp.where(s > 0.0, 1.0 / s, 0.0)
    u = m * inv_s
    corr = (gram * inv_s[:, None] * inv_s[None, :]
            - T * u[:, None] * u[None, :])
    adj = jnp.abs(jnp.clip(corr, -1.0, 1.0))

    rs1 = jnp.sum(adj, axis=1, keepdims=True)
    rs2 = jnp.sum(adj * adj, axis=1, keepdims=True)
    inv_rn = 1.0 / (jnp.sqrt(rs2) + EPS)
    deg = rs1 * inv_rn + 1.0
    dinv = jax.lax.rsqrt(deg)

    adjb = adj.astype(jnp.bfloat16)

    def layer(sup):
        supd = sup * dinv
        agg = inv_rn * jnp.dot(adjb, supd.astype(jnp.bfloat16),
                               preferred_element_type=jnp.float32)
        return jnp.maximum((agg + supd) * dinv, 0.0)

    w1b = w1_ref[...].astype(jnp.bfloat16)
    s1 = inv_rn * jnp.dot(adjb, w1b, preferred_element_type=jnp.float32)
    x1 = layer(s1 + b1_ref[...])
    s2 = jnp.dot(x1, w2_ref[...], preferred_element_type=jnp.float32)
    out_ref[0] = layer(s2 + b2_ref[...])


@jax.jit
def kernel(fMRI, W1, b1, W2, b2):
    b1r = b1.reshape(1, H)
    b2r = b2.reshape(1, H)
    out = pl.pallas_call(
        _gcn_block_kernel,
        grid=(B,),
        in_specs=[
            pl.BlockSpec((1, T, N), lambda b: (b, 0, 0)),
            pl.BlockSpec((N, H), lambda b: (0, 0)),
            pl.BlockSpec((1, H), lambda b: (0, 0)),
            pl.BlockSpec((H, H), lambda b: (0, 0)),
            pl.BlockSpec((1, H), lambda b: (0, 0)),
        ],
        out_specs=pl.BlockSpec((1, N, H), lambda b: (b, 0, 0)),
        out_shape=jax.ShapeDtypeStruct((B, N, H), jnp.float32),
    )(fMRI, W1, b1r, W2, b2r)
    return out


# speedup vs baseline: 8.3193x; 1.0522x over previous
import jax
import jax.numpy as jnp
from jax.experimental import pallas as pl

B, T, N, H = 8, 512, 400, 128
EPS = 1e-8


def _gcn_block_kernel(fmri_ref, w1_ref, b1_ref, w2_ref, b2_ref, out_ref):
    xb = fmri_ref[0]
    gram = jax.lax.dot_general(
        xb, xb, (((0,), (0,)), ((), ())),
        preferred_element_type=jnp.float32)
    x = xb.astype(jnp.float32)
    colsum = jnp.sum(x, axis=0)
    sumsq = jnp.sum(x * x, axis=0)
    m = colsum * (1.0 / T)
    var = sumsq - T * m * m
    s = jnp.sqrt(var)
    inv_s = jnp.where(s > 0.0, 1.0 / s, 0.0)
    u = m * inv_s
    corr = (gram * inv_s[:, None] * inv_s[None, :]
            - T * u[:, None] * u[None, :])
    adj = jnp.abs(jnp.clip(corr, -1.0, 1.0))

    rs1 = jnp.sum(adj, axis=1, keepdims=True)
    rs2 = jnp.sum(adj * adj, axis=1, keepdims=True)
    inv_rn = 1.0 / (jnp.sqrt(rs2) + EPS)
    deg = rs1 * inv_rn + 1.0
    dinv = jax.lax.rsqrt(deg)

    adjb = adj.astype(jnp.bfloat16)

    def layer(sup):
        supd = sup * dinv
        agg = inv_rn * jnp.dot(adjb, supd.astype(jnp.bfloat16),
                               preferred_element_type=jnp.float32)
        return jnp.maximum((agg + supd) * dinv, 0.0)

    w1b = w1_ref[...].astype(jnp.bfloat16)
    s1 = inv_rn * jnp.dot(adjb, w1b, preferred_element_type=jnp.float32)
    x1 = layer(s1 + b1_ref[...])
    s2 = jnp.dot(x1, w2_ref[...], preferred_element_type=jnp.float32)
    out_ref[0] = layer(s2 + b2_ref[...])


@jax.jit
def kernel(fMRI, W1, b1, W2, b2):
    fMRIb = fMRI.astype(jnp.bfloat16)
    b1r = b1.reshape(1, H)
    b2r = b2.reshape(1, H)
    out = pl.pallas_call(
        _gcn_block_kernel,
        grid=(B,),
        in_specs=[
            pl.BlockSpec((1, T, N), lambda b: (b, 0, 0)),
            pl.BlockSpec((N, H), lambda b: (0, 0)),
            pl.BlockSpec((1, H), lambda b: (0, 0)),
            pl.BlockSpec((H, H), lambda b: (0, 0)),
            pl.BlockSpec((1, H), lambda b: (0, 0)),
        ],
        out_specs=pl.BlockSpec((1, N, H), lambda b: (b, 0, 0)),
        out_shape=jax.ShapeDtypeStruct((B, N, H), jnp.float32),
    )(fMRIb, W1, b1r, W2, b2r)
    return out
